# baseline (speedup 1.0000x reference)
import jax
import jax.numpy as jnp
from jax import lax
from jax.experimental import pallas as pl
from jax.experimental.pallas import tpu as pltpu
from jax.experimental.pallas import tpu_sc as plsc

N = 10000
E = 320000
G = 16
DN, DE, DU = 128, 16, 64
OE, ON, OU = 128, 128, 64

NC, NS = 2, 16
HF = OE // NC
EPT = E // NS
CH = 80
MB = 4000
NMB = EPT // MB
MCH = MB // CH
AR = 10112
ACW = HF + 16
RPT = AR // NS
ZR = 80


def _node_pre_body(x_ref, b2_ref, u_ref, we_ref, be_ref, out_ref):
    we = we_ref[...]
    uw = jnp.dot(u_ref[...], we[DN + DE:], preferred_element_type=jnp.float32)
    oh = (b2_ref[...] == lax.broadcasted_iota(jnp.int32, (1, G), 1)
          ).astype(jnp.float32)
    np_full = (jnp.dot(x_ref[...], we[:DN], preferred_element_type=jnp.float32)
               + jnp.dot(oh, uw, preferred_element_type=jnp.float32)
               + be_ref[...])
    out_ref[0] = np_full[:, :HF]
    out_ref[1] = np_full[:, HF:]


def _eat_body(attr_ref, we_ref, out_ref):
    out_ref[...] = jnp.dot(attr_ref[...], we_ref[DN:DN + DE],
                           preferred_element_type=jnp.float32)


def _sc_edge_body(node_pre_hbm, eat_hbm, src_hbm, dst_hbm, batch_hbm,
                  ea_hbm, acc_hbm, gacc_hbm,
                  batch_v, srcall_v, dstall_v,
                  src_v0, src_v1, dst_v0, dst_v1, gsrc_v0, gsrc_v1,
                  rows_v0, rows_v1, eat_v0, eat_v1, out_v0, out_v1,
                  zrow_v, acc_sh, graph_sh,
                  sem_g0, sem_g1, sem_e0, sem_e1, sem_o0, sem_o1):
    src_v = [src_v0, src_v1]
    dst_v = [dst_v0, dst_v1]
    gsrc_v = [gsrc_v0, gsrc_v1]
    rows_v = [rows_v0, rows_v1]
    eat_v = [eat_v0, eat_v1]
    out_v = [out_v0, out_v1]
    sem_g = [sem_g0, sem_g1]
    sem_e = [sem_e0, sem_e1]
    sem_o = [sem_o0, sem_o1]
    cid = lax.axis_index("c")
    sid = lax.axis_index("s")

    fz = jnp.zeros((16,), jnp.float32)
    lane = lax.iota(jnp.int32, 16)
    cnt_vec = jnp.where(lane == 0, 1.0, 0.0).astype(jnp.float32)

    def _z1(i, c):
        for j in range(ACW // 16):
            zrow_v[i, pl.ds(j * 16, 16)] = fz
        return c
    lax.fori_loop(0, ZR, _z1, 0)

    zbase = sid * RPT

    def _z2(k, c):
        pltpu.sync_copy(zrow_v, acc_sh.at[pl.ds(zbase + k * ZR, ZR)])
        return c
    lax.fori_loop(0, RPT // ZR, _z2, 0)
    rem = RPT % ZR
    pltpu.sync_copy(zrow_v.at[pl.ds(0, rem)],
                    acc_sh.at[pl.ds(zbase + (RPT // ZR) * ZR, rem)])

    pltpu.sync_copy(zrow_v.at[pl.ds(0, G)], graph_sh.at[sid])

    tbase = pl.multiple_of(sid * EPT, 8)
    pltpu.sync_copy(batch_hbm, batch_v)

    def _cinit(i, c):
        for b in range(2):
            out_v[b][i, pl.ds(HF, 16)] = cnt_vec
        return c
    lax.fori_loop(0, CH, _cinit, 0)
    plsc.subcore_barrier()

    def _stage_in(mbase, t, b):
        toff = t * CH

        def _cp(k, cc):
            src_v[b][pl.ds(k * 16, 16)] = srcall_v[pl.ds(toff + k * 16, 16)]
            return cc
        lax.fori_loop(0, CH // 16, _cp, 0)
        ebase = pl.multiple_of(mbase + t * CH, 8)
        pltpu.async_copy(node_pre_hbm.at[cid].at[src_v[b]], rows_v[b],
                         sem_g[b])
        pltpu.async_copy(eat_hbm.at[pl.ds(ebase, CH), pl.ds(cid * HF, HF)],
                         eat_v[b], sem_e[b])

    def _step(mbase, t, b):
        nb = 1 - b
        ebase = pl.multiple_of(mbase + t * CH, 8)
        pltpu.make_async_copy(node_pre_hbm.at[cid].at[src_v[b]],
                              rows_v[b], sem_g[b]).wait()
        pltpu.make_async_copy(
            eat_hbm.at[pl.ds(ebase, CH), pl.ds(cid * HF, HF)],
            eat_v[b], sem_e[b]).wait()

        @pl.when(t + 1 < MCH)
        def _():
            _stage_in(mbase, t + 1, nb)

        @pl.when(t >= 2)
        def _():
            eb2 = pl.multiple_of(mbase + (t - 2) * CH, 8)
            pltpu.make_async_copy(
                out_v[b].at[:, pl.ds(0, HF)],
                ea_hbm.at[pl.ds(eb2, CH), pl.ds(cid * HF, HF)],
                sem_o[b]).wait()

        toff = t * CH

        def _cpd(k, cc):
            dv = dstall_v[pl.ds(toff + k * 16, 16)]
            dst_v[b][pl.ds(k * 16, 16)] = dv
            sv = src_v[b][pl.ds(k * 16, 16)]
            gsrc_v[b][pl.ds(k * 16, 16)] = plsc.load_gather(batch_v, [sv])
            return cc
        lax.fori_loop(0, CH // 16, _cpd, 0)

        def _row(i, cc):
            for j in range(HF // 16):
                v = (rows_v[b][i, pl.ds(j * 16, 16)]
                     + eat_v[b][i, pl.ds(j * 16, 16)])
                out_v[b][i, pl.ds(j * 16, 16)] = jnp.maximum(v, 0.0)
            return cc
        lax.fori_loop(0, CH, _row, 0)

        pltpu.async_copy(out_v[b].at[:, pl.ds(0, HF)],
                         ea_hbm.at[pl.ds(ebase, CH), pl.ds(cid * HF, HF)],
                         sem_o[b])
        pltpu.sync_copy(out_v[b], acc_sh.at[dst_v[b]], add=True)
        pltpu.sync_copy(out_v[b], graph_sh.at[sid].at[gsrc_v[b]], add=True)

    def _meta(m, c):
        mbase = pl.multiple_of(tbase + m * MB, 8)
        pltpu.sync_copy(src_hbm.at[pl.ds(mbase, MB)], srcall_v)
        pltpu.sync_copy(dst_hbm.at[pl.ds(mbase, MB)], dstall_v)
        _stage_in(mbase, 0, 0)

        def _pair(k, cc):
            _step(mbase, 2 * k, 0)
            _step(mbase, 2 * k + 1, 1)
            return cc
        lax.fori_loop(0, MCH // 2, _pair, 0)
        for (tt, bb) in ((MCH - 2, 0), (MCH - 1, 1)):
            ebt = pl.multiple_of(mbase + tt * CH, 8)
            pltpu.make_async_copy(
                out_v[bb].at[:, pl.ds(0, HF)],
                ea_hbm.at[pl.ds(ebt, CH), pl.ds(cid * HF, HF)],
                sem_o[bb]).wait()
        return c

    lax.fori_loop(0, NMB, _meta, 0)

    plsc.subcore_barrier()
    pltpu.sync_copy(acc_sh.at[pl.ds(sid * RPT, RPT)],
                    acc_hbm.at[cid, pl.ds(sid * RPT, RPT)])

    pltpu.sync_copy(graph_sh.at[sid], gacc_hbm.at[cid, sid])


def _build_sc_edge():
    mesh = plsc.VectorSubcoreMesh(core_axis_name="c", subcore_axis_name="s")
    return pl.kernel(
        _sc_edge_body,
        out_type=(
            jax.ShapeDtypeStruct((E, OE), jnp.float32),
            jax.ShapeDtypeStruct((NC, AR, ACW), jnp.float32),
            jax.ShapeDtypeStruct((NC, NS, G, ACW), jnp.float32),
        ),
        mesh=mesh,
        compiler_params=pltpu.CompilerParams(use_tc_tiling_on_sc=False,
                                             needs_layout_passes=False),
        scratch_types=(
            [
                pltpu.VMEM((N,), jnp.int32),
                pltpu.VMEM((MB,), jnp.int32),
                pltpu.VMEM((MB,), jnp.int32),
            ]
            + [pltpu.VMEM((CH,), jnp.int32)] * 6
            + [pltpu.VMEM((CH, HF), jnp.float32)] * 4
            + [pltpu.VMEM((CH, ACW), jnp.float32)] * 2
            + [
                pltpu.VMEM((ZR, ACW), jnp.float32),
                pltpu.VMEM_SHARED((AR, ACW), jnp.float32),
                pltpu.VMEM_SHARED((NS, G, ACW), jnp.float32),
            ]
            + [pltpu.SemaphoreType.DMA] * 6
        ),
    )


def _final_body(x_ref, b2_ref, u_ref, acc0_ref, acc1_ref, gacc_ref,
                wn_ref, bn_ref, wg_ref, bg_ref, xn_ref, un_ref):
    acc0 = acc0_ref[...]
    acc1 = acc1_ref[...]
    node_cnt = jnp.maximum(acc0[:N, HF:HF + 1], 1.0)
    msg0 = acc0[:N, :HF] / node_cnt
    msg1 = acc1[:N, :HF] / node_cnt

    wn = wn_ref[...]
    u_full = u_ref[...]
    uwn = jnp.dot(u_full, wn[DN + OE:], preferred_element_type=jnp.float32)
    oh = (b2_ref[...] == lax.broadcasted_iota(jnp.int32, (1, G), 1)
          ).astype(jnp.float32)
    xn = (jnp.dot(x_ref[...], wn[:DN], preferred_element_type=jnp.float32)
          + jnp.dot(msg0, wn[DN:DN + HF], preferred_element_type=jnp.float32)
          + jnp.dot(msg1, wn[DN + HF:DN + OE],
                    preferred_element_type=jnp.float32)
          + jnp.dot(oh, uwn, preferred_element_type=jnp.float32)
          + bn_ref[...])
    xn = jnp.maximum(xn, 0.0)
    xn_ref[...] = xn

    gsum = lax.dot_general(oh, xn, (((0,), (0,)), ((), ())),
                           preferred_element_type=jnp.float32)
    gcnt = lax.dot_general(oh, jnp.ones((N, 1), jnp.float32),
                           (((0,), (0,)), ((), ())),
                           preferred_element_type=jnp.float32)
    x_global = gsum / jnp.maximum(gcnt, 1.0)

    gacc0 = jnp.sum(gacc_ref[0], axis=0)
    gacc1 = jnp.sum(gacc_ref[1], axis=0)
    eg_cnt = jnp.maximum(gacc0[:, HF:HF + 1], 1.0)
    eg0 = gacc0[:, :HF] / eg_cnt
    eg1 = gacc1[:, :HF] / eg_cnt

    wg = wg_ref[...]
    un = (jnp.dot(eg0, wg[:HF], preferred_element_type=jnp.float32)
          + jnp.dot(eg1, wg[HF:ON], preferred_element_type=jnp.float32)
          + jnp.dot(x_global, wg[ON:ON + OE], preferred_element_type=jnp.float32)
          + jnp.dot(u_full, wg[ON + OE:], preferred_element_type=jnp.float32)
          + bg_ref[...])
    un_ref[...] = jnp.maximum(un, 0.0)


@jax.jit
def kernel(x, edge_attr, edge_index, u, batch, W_e, b_e, W_n, b_n, W_g, b_g):
    batch2d = batch.reshape(N, 1)
    src = edge_index[0]
    dst = edge_index[1]

    node_pre = pl.pallas_call(
        _node_pre_body,
        out_shape=jax.ShapeDtypeStruct((NC, N, HF), jnp.float32),
    )(x, batch2d, u, W_e, b_e)

    BE = 8000
    eat = pl.pallas_call(
        _eat_body,
        grid=(E // BE,),
        in_specs=[
            pl.BlockSpec((BE, DE), lambda i: (i, 0)),
            pl.BlockSpec((DN + DE + DU, OE), lambda i: (0, 0)),
        ],
        out_specs=pl.BlockSpec((BE, OE), lambda i: (i, 0)),
        out_shape=jax.ShapeDtypeStruct((E, OE), jnp.float32),
    )(edge_attr, W_e)

    ea, acc, gacc = _build_sc_edge()(node_pre, eat, src, dst, batch)

    xn, un = pl.pallas_call(
        _final_body,
        out_shape=(
            jax.ShapeDtypeStruct((N, ON), jnp.float32),
            jax.ShapeDtypeStruct((G, OU), jnp.float32),
        ),
    )(x, batch2d, u, acc[0], acc[1], gacc, W_n, b_n, W_g, b_g)

    return (xn, ea, edge_index, un, batch)


# speedup vs baseline: 1.0095x; 1.0095x over previous
import jax
import jax.numpy as jnp
from jax import lax
from jax.experimental import pallas as pl
from jax.experimental.pallas import tpu as pltpu
from jax.experimental.pallas import tpu_sc as plsc

N = 10000
E = 320000
G = 16
DN, DE, DU = 128, 16, 64
OE, ON, OU = 128, 128, 64

NC, NS = 2, 16
HF = OE // NC
EPT = E // NS
CH = 80
MB = 4000
NMB = EPT // MB
MCH = MB // CH
AR = 10112
ACW = HF + 16
RPT = AR // NS
ZR = 80


def _node_pre_body(x_ref, b2_ref, u_ref, we_ref, be_ref, out_ref):
    we = we_ref[...]
    uw = jnp.dot(u_ref[...], we[DN + DE:], preferred_element_type=jnp.float32)
    oh = (b2_ref[...] == lax.broadcasted_iota(jnp.int32, (1, G), 1)
          ).astype(jnp.float32)
    np_full = (jnp.dot(x_ref[...], we[:DN], preferred_element_type=jnp.float32)
               + jnp.dot(oh, uw, preferred_element_type=jnp.float32)
               + be_ref[...])
    out_ref[0] = np_full[:, :HF]
    out_ref[1] = np_full[:, HF:]


def _eat_body(attr_ref, we_ref, out_ref):
    out_ref[...] = jnp.dot(attr_ref[...], we_ref[DN:DN + DE],
                           preferred_element_type=jnp.float32)


def _sc_edge_body(node_pre_hbm, eat_hbm, src_hbm, dst_hbm, batch_hbm,
                  ea_hbm, acc_hbm, gacc_hbm,
                  batch_v, srcall_v, dstall_v,
                  src_v0, src_v1, dst_v0, dst_v1, gsrc_v0, gsrc_v1,
                  rows_v0, rows_v1, eat_v0, eat_v1, out_v0, out_v1,
                  zrow_v, acc_sh, graph_sh,
                  sem_g0, sem_g1, sem_e0, sem_e1, sem_o0, sem_o1,
                  sem_a0, sem_a1):
    src_v = [src_v0, src_v1]
    dst_v = [dst_v0, dst_v1]
    gsrc_v = [gsrc_v0, gsrc_v1]
    rows_v = [rows_v0, rows_v1]
    eat_v = [eat_v0, eat_v1]
    out_v = [out_v0, out_v1]
    sem_g = [sem_g0, sem_g1]
    sem_e = [sem_e0, sem_e1]
    sem_o = [sem_o0, sem_o1]
    sem_a = [sem_a0, sem_a1]
    cid = lax.axis_index("c")
    sid = lax.axis_index("s")

    fz = jnp.zeros((16,), jnp.float32)
    lane = lax.iota(jnp.int32, 16)
    cnt_vec = jnp.where(lane == 0, 1.0, 0.0).astype(jnp.float32)

    def _z1(i, c):
        for j in range(ACW // 16):
            zrow_v[i, pl.ds(j * 16, 16)] = fz
        return c
    lax.fori_loop(0, ZR, _z1, 0)

    zbase = sid * RPT

    def _z2(k, c):
        pltpu.sync_copy(zrow_v, acc_sh.at[pl.ds(zbase + k * ZR, ZR)])
        return c
    lax.fori_loop(0, RPT // ZR, _z2, 0)
    rem = RPT % ZR
    pltpu.sync_copy(zrow_v.at[pl.ds(0, rem)],
                    acc_sh.at[pl.ds(zbase + (RPT // ZR) * ZR, rem)])

    pltpu.sync_copy(zrow_v.at[pl.ds(0, G)], graph_sh.at[sid])

    tbase = pl.multiple_of(sid * EPT, 8)
    pltpu.sync_copy(batch_hbm, batch_v)

    def _cinit(i, c):
        for b in range(2):
            out_v[b][i, pl.ds(HF, 16)] = cnt_vec
        return c
    lax.fori_loop(0, CH, _cinit, 0)
    plsc.subcore_barrier()

    def _stage_in(mbase, t, b):
        toff = t * CH

        def _cp(k, cc):
            src_v[b][pl.ds(k * 16, 16)] = srcall_v[pl.ds(toff + k * 16, 16)]
            return cc
        lax.fori_loop(0, CH // 16, _cp, 0)
        ebase = pl.multiple_of(mbase + t * CH, 8)
        pltpu.async_copy(node_pre_hbm.at[cid].at[src_v[b]], rows_v[b],
                         sem_g[b])
        pltpu.async_copy(eat_hbm.at[pl.ds(ebase, CH), pl.ds(cid * HF, HF)],
                         eat_v[b], sem_e[b])

    def _step(mbase, t, b):
        nb = 1 - b
        ebase = pl.multiple_of(mbase + t * CH, 8)
        pltpu.make_async_copy(node_pre_hbm.at[cid].at[src_v[b]],
                              rows_v[b], sem_g[b]).wait()
        pltpu.make_async_copy(
            eat_hbm.at[pl.ds(ebase, CH), pl.ds(cid * HF, HF)],
            eat_v[b], sem_e[b]).wait()

        @pl.when(t + 1 < MCH)
        def _():
            _stage_in(mbase, t + 1, nb)

        @pl.when(t >= 2)
        def _():
            eb2 = pl.multiple_of(mbase + (t - 2) * CH, 8)
            pltpu.make_async_copy(
                out_v[b].at[:, pl.ds(0, HF)],
                ea_hbm.at[pl.ds(eb2, CH), pl.ds(cid * HF, HF)],
                sem_o[b]).wait()
            pltpu.make_async_copy(out_v[b], acc_sh.at[dst_v[b]],
                                  sem_a[b]).wait()

        toff = t * CH

        def _cpd(k, cc):
            dv = dstall_v[pl.ds(toff + k * 16, 16)]
            dst_v[b][pl.ds(k * 16, 16)] = dv
            sv = src_v[b][pl.ds(k * 16, 16)]
            gsrc_v[b][pl.ds(k * 16, 16)] = plsc.load_gather(batch_v, [sv])
            return cc
        lax.fori_loop(0, CH // 16, _cpd, 0)

        def _row(q, cc):
            for r in range(2):
                i = 2 * q + r
                for j in range(HF // 16):
                    v = (rows_v[b][i, pl.ds(j * 16, 16)]
                         + eat_v[b][i, pl.ds(j * 16, 16)])
                    out_v[b][i, pl.ds(j * 16, 16)] = jnp.maximum(v, 0.0)
            return cc
        lax.fori_loop(0, CH // 2, _row, 0)

        pltpu.async_copy(out_v[b].at[:, pl.ds(0, HF)],
                         ea_hbm.at[pl.ds(ebase, CH), pl.ds(cid * HF, HF)],
                         sem_o[b])
        pltpu.async_copy(out_v[b], acc_sh.at[dst_v[b]], sem_a[b], add=True)
        pltpu.sync_copy(out_v[b], graph_sh.at[sid].at[gsrc_v[b]], add=True)

    def _meta(m, c):
        mbase = pl.multiple_of(tbase + m * MB, 8)
        pltpu.sync_copy(src_hbm.at[pl.ds(mbase, MB)], srcall_v)
        pltpu.sync_copy(dst_hbm.at[pl.ds(mbase, MB)], dstall_v)
        _stage_in(mbase, 0, 0)

        def _pair(k, cc):
            _step(mbase, 2 * k, 0)
            _step(mbase, 2 * k + 1, 1)
            return cc
        lax.fori_loop(0, MCH // 2, _pair, 0)
        for (tt, bb) in ((MCH - 2, 0), (MCH - 1, 1)):
            ebt = pl.multiple_of(mbase + tt * CH, 8)
            pltpu.make_async_copy(
                out_v[bb].at[:, pl.ds(0, HF)],
                ea_hbm.at[pl.ds(ebt, CH), pl.ds(cid * HF, HF)],
                sem_o[bb]).wait()
            pltpu.make_async_copy(out_v[bb], acc_sh.at[dst_v[bb]],
                                  sem_a[bb]).wait()
        return c

    lax.fori_loop(0, NMB, _meta, 0)

    plsc.subcore_barrier()
    pltpu.sync_copy(acc_sh.at[pl.ds(sid * RPT, RPT)],
                    acc_hbm.at[cid, pl.ds(sid * RPT, RPT)])

    pltpu.sync_copy(graph_sh.at[sid], gacc_hbm.at[cid, sid])


def _build_sc_edge():
    mesh = plsc.VectorSubcoreMesh(core_axis_name="c", subcore_axis_name="s")
    return pl.kernel(
        _sc_edge_body,
        out_type=(
            jax.ShapeDtypeStruct((E, OE), jnp.float32),
            jax.ShapeDtypeStruct((NC, AR, ACW), jnp.float32),
            jax.ShapeDtypeStruct((NC, NS, G, ACW), jnp.float32),
        ),
        mesh=mesh,
        compiler_params=pltpu.CompilerParams(use_tc_tiling_on_sc=False,
                                             needs_layout_passes=False),
        scratch_types=(
            [
                pltpu.VMEM((N,), jnp.int32),
                pltpu.VMEM((MB,), jnp.int32),
                pltpu.VMEM((MB,), jnp.int32),
            ]
            + [pltpu.VMEM((CH,), jnp.int32)] * 6
            + [pltpu.VMEM((CH, HF), jnp.float32)] * 4
            + [pltpu.VMEM((CH, ACW), jnp.float32)] * 2
            + [
                pltpu.VMEM((ZR, ACW), jnp.float32),
                pltpu.VMEM_SHARED((AR, ACW), jnp.float32),
                pltpu.VMEM_SHARED((NS, G, ACW), jnp.float32),
            ]
            + [pltpu.SemaphoreType.DMA] * 8
        ),
    )


def _final_body(x_ref, b2_ref, u_ref, acc0_ref, acc1_ref, gacc_ref,
                wn_ref, bn_ref, wg_ref, bg_ref, xn_ref, un_ref):
    acc0 = acc0_ref[...]
    acc1 = acc1_ref[...]
    node_cnt = jnp.maximum(acc0[:N, HF:HF + 1], 1.0)
    msg0 = acc0[:N, :HF] / node_cnt
    msg1 = acc1[:N, :HF] / node_cnt

    wn = wn_ref[...]
    u_full = u_ref[...]
    uwn = jnp.dot(u_full, wn[DN + OE:], preferred_element_type=jnp.float32)
    oh = (b2_ref[...] == lax.broadcasted_iota(jnp.int32, (1, G), 1)
          ).astype(jnp.float32)
    xn = (jnp.dot(x_ref[...], wn[:DN], preferred_element_type=jnp.float32)
          + jnp.dot(msg0, wn[DN:DN + HF], preferred_element_type=jnp.float32)
          + jnp.dot(msg1, wn[DN + HF:DN + OE],
                    preferred_element_type=jnp.float32)
          + jnp.dot(oh, uwn, preferred_element_type=jnp.float32)
          + bn_ref[...])
    xn = jnp.maximum(xn, 0.0)
    xn_ref[...] = xn

    gsum = lax.dot_general(oh, xn, (((0,), (0,)), ((), ())),
                           preferred_element_type=jnp.float32)
    gcnt = lax.dot_general(oh, jnp.ones((N, 1), jnp.float32),
                           (((0,), (0,)), ((), ())),
                           preferred_element_type=jnp.float32)
    x_global = gsum / jnp.maximum(gcnt, 1.0)

    gacc0 = jnp.sum(gacc_ref[0], axis=0)
    gacc1 = jnp.sum(gacc_ref[1], axis=0)
    eg_cnt = jnp.maximum(gacc0[:, HF:HF + 1], 1.0)
    eg0 = gacc0[:, :HF] / eg_cnt
    eg1 = gacc1[:, :HF] / eg_cnt

    wg = wg_ref[...]
    un = (jnp.dot(eg0, wg[:HF], preferred_element_type=jnp.float32)
          + jnp.dot(eg1, wg[HF:ON], preferred_element_type=jnp.float32)
          + jnp.dot(x_global, wg[ON:ON + OE], preferred_element_type=jnp.float32)
          + jnp.dot(u_full, wg[ON + OE:], preferred_element_type=jnp.float32)
          + bg_ref[...])
    un_ref[...] = jnp.maximum(un, 0.0)


@jax.jit
def kernel(x, edge_attr, edge_index, u, batch, W_e, b_e, W_n, b_n, W_g, b_g):
    batch2d = batch.reshape(N, 1)
    src = edge_index[0]
    dst = edge_index[1]

    node_pre = pl.pallas_call(
        _node_pre_body,
        out_shape=jax.ShapeDtypeStruct((NC, N, HF), jnp.float32),
    )(x, batch2d, u, W_e, b_e)

    BE = 8000
    eat = pl.pallas_call(
        _eat_body,
        grid=(E // BE,),
        in_specs=[
            pl.BlockSpec((BE, DE), lambda i: (i, 0)),
            pl.BlockSpec((DN + DE + DU, OE), lambda i: (0, 0)),
        ],
        out_specs=pl.BlockSpec((BE, OE), lambda i: (i, 0)),
        out_shape=jax.ShapeDtypeStruct((E, OE), jnp.float32),
    )(edge_attr, W_e)

    ea, acc, gacc = _build_sc_edge()(node_pre, eat, src, dst, batch)

    xn, un = pl.pallas_call(
        _final_body,
        out_shape=(
            jax.ShapeDtypeStruct((N, ON), jnp.float32),
            jax.ShapeDtypeStruct((G, OU), jnp.float32),
        ),
    )(x, batch2d, u, acc[0], acc[1], gacc, W_n, b_n, W_g, b_g)

    return (xn, ea, edge_index, un, batch)


# speedup vs baseline: 1.1512x; 1.1404x over previous
import jax
import jax.numpy as jnp
from jax import lax
from jax.experimental import pallas as pl
from jax.experimental.pallas import tpu as pltpu
from jax.experimental.pallas import tpu_sc as plsc

N = 10000
E = 320000
G = 16
DN, DE, DU = 128, 16, 64
OE, ON, OU = 128, 128, 64

NC, NS = 2, 16
HF = OE // NC
EPT = E // NS
CH = 80
MB = 4000
NMB = EPT // MB
MCH = MB // CH
AR = 10112
ACW = HF + 16
RPT = AR // NS
ZR = 80


def _node_pre_body(x_ref, b2_ref, u_ref, we_ref, be_ref, out_ref):
    we = we_ref[...]
    uw = jnp.dot(u_ref[...], we[DN + DE:], preferred_element_type=jnp.float32)
    oh = (b2_ref[...] == lax.broadcasted_iota(jnp.int32, (1, G), 1)
          ).astype(jnp.float32)
    np_full = (jnp.dot(x_ref[...], we[:DN], preferred_element_type=jnp.float32)
               + jnp.dot(oh, uw, preferred_element_type=jnp.float32)
               + be_ref[...])
    out_ref[0] = np_full[:, :HF]
    out_ref[1] = np_full[:, HF:]


def _eat_body(attr_ref, we_ref, out_ref):
    out_ref[...] = jnp.dot(attr_ref[...], we_ref[DN:DN + DE],
                           preferred_element_type=jnp.float32)


def _sc_edge_body(node_pre_hbm, eat_hbm, src_hbm, dst_hbm, batch_hbm,
                  ea_hbm, acc_hbm, gacc_hbm,
                  batch_v, srcall_v, dstall_v,
                  src_v0, src_v1, dst_v0, dst_v1, gdst_v0, gdst_v1,
                  rows_v0, rows_v1, eat_v0, eat_v1, out_v0, out_v1,
                  zrow_v, acc_sh, gacc_sh,
                  sem_g0, sem_g1, sem_e0, sem_e1, sem_o0, sem_o1,
                  sem_a0, sem_a1, sem_ga0, sem_ga1):
    src_v = [src_v0, src_v1]
    dst_v = [dst_v0, dst_v1]
    gdst_v = [gdst_v0, gdst_v1]
    rows_v = [rows_v0, rows_v1]
    eat_v = [eat_v0, eat_v1]
    out_v = [out_v0, out_v1]
    sem_g = [sem_g0, sem_g1]
    sem_e = [sem_e0, sem_e1]
    sem_o = [sem_o0, sem_o1]
    sem_a = [sem_a0, sem_a1]
    sem_ga = [sem_ga0, sem_ga1]
    cid = lax.axis_index("c")
    sid = lax.axis_index("s")

    fz = jnp.zeros((16,), jnp.float32)
    lane = lax.iota(jnp.int32, 16)
    cnt_vec = jnp.where(lane == 0, 1.0, 0.0).astype(jnp.float32)

    def _z1(i, c):
        for j in range(ACW // 16):
            zrow_v[i, pl.ds(j * 16, 16)] = fz
        return c
    lax.fori_loop(0, ZR, _z1, 0)

    zbase = sid * RPT

    def _z2(k, c):
        pltpu.sync_copy(zrow_v, acc_sh.at[pl.ds(zbase + k * ZR, ZR)])
        return c
    lax.fori_loop(0, RPT // ZR, _z2, 0)
    rem = RPT % ZR
    pltpu.sync_copy(zrow_v.at[pl.ds(0, rem)],
                    acc_sh.at[pl.ds(zbase + (RPT // ZR) * ZR, rem)])
    pltpu.sync_copy(zrow_v.at[pl.ds(0, G)], gacc_sh.at[pl.ds(sid * G, G)])

    tbase = pl.multiple_of(sid * EPT, 8)
    pltpu.sync_copy(batch_hbm, batch_v)

    def _cinit(i, c):
        for b in range(2):
            out_v[b][i, pl.ds(HF, 16)] = cnt_vec
        return c
    lax.fori_loop(0, CH, _cinit, 0)
    plsc.subcore_barrier()

    def _stage_in(mbase, t, b):
        toff = t * CH

        def _cp(k, cc):
            src_v[b][pl.ds(k * 16, 16)] = srcall_v[pl.ds(toff + k * 16, 16)]
            return cc
        lax.fori_loop(0, CH // 16, _cp, 0)
        ebase = pl.multiple_of(mbase + t * CH, 8)
        pltpu.async_copy(node_pre_hbm.at[cid].at[src_v[b]], rows_v[b],
                         sem_g[b])
        pltpu.async_copy(eat_hbm.at[pl.ds(ebase, CH), pl.ds(cid * HF, HF)],
                         eat_v[b], sem_e[b])

    def _step(mbase, t, b):
        nb = 1 - b
        ebase = pl.multiple_of(mbase + t * CH, 8)
        pltpu.make_async_copy(node_pre_hbm.at[cid].at[src_v[b]],
                              rows_v[b], sem_g[b]).wait()
        pltpu.make_async_copy(
            eat_hbm.at[pl.ds(ebase, CH), pl.ds(cid * HF, HF)],
            eat_v[b], sem_e[b]).wait()

        @pl.when(t + 1 < MCH)
        def _():
            _stage_in(mbase, t + 1, nb)

        @pl.when(t >= 2)
        def _():
            eb2 = pl.multiple_of(mbase + (t - 2) * CH, 8)
            pltpu.make_async_copy(
                out_v[b].at[:, pl.ds(0, HF)],
                ea_hbm.at[pl.ds(eb2, CH), pl.ds(cid * HF, HF)],
                sem_o[b]).wait()
            pltpu.make_async_copy(out_v[b], acc_sh.at[dst_v[b]],
                                  sem_a[b]).wait()
            pltpu.make_async_copy(out_v[b], gacc_sh.at[gdst_v[b]],
                                  sem_ga[b]).wait()

        toff = t * CH

        def _cpd(k, cc):
            dv = dstall_v[pl.ds(toff + k * 16, 16)]
            dst_v[b][pl.ds(k * 16, 16)] = dv
            sv = src_v[b][pl.ds(k * 16, 16)]
            gv = plsc.load_gather(batch_v, [sv])
            gdst_v[b][pl.ds(k * 16, 16)] = gv + sid * G
            return cc
        lax.fori_loop(0, CH // 16, _cpd, 0)

        def _row(q, cc):
            for r in range(2):
                i = 2 * q + r
                for j in range(HF // 16):
                    v = (rows_v[b][i, pl.ds(j * 16, 16)]
                         + eat_v[b][i, pl.ds(j * 16, 16)])
                    out_v[b][i, pl.ds(j * 16, 16)] = jnp.maximum(v, 0.0)
            return cc
        lax.fori_loop(0, CH // 2, _row, 0)

        pltpu.async_copy(out_v[b].at[:, pl.ds(0, HF)],
                         ea_hbm.at[pl.ds(ebase, CH), pl.ds(cid * HF, HF)],
                         sem_o[b])
        pltpu.async_copy(out_v[b], acc_sh.at[dst_v[b]], sem_a[b], add=True)
        pltpu.async_copy(out_v[b], gacc_sh.at[gdst_v[b]], sem_ga[b], add=True)

    def _meta(m, c):
        mbase = pl.multiple_of(tbase + m * MB, 8)
        pltpu.sync_copy(src_hbm.at[pl.ds(mbase, MB)], srcall_v)
        pltpu.sync_copy(dst_hbm.at[pl.ds(mbase, MB)], dstall_v)
        _stage_in(mbase, 0, 0)

        def _pair(k, cc):
            _step(mbase, 2 * k, 0)
            _step(mbase, 2 * k + 1, 1)
            return cc
        lax.fori_loop(0, MCH // 2, _pair, 0)
        for (tt, bb) in ((MCH - 2, 0), (MCH - 1, 1)):
            ebt = pl.multiple_of(mbase + tt * CH, 8)
            pltpu.make_async_copy(
                out_v[bb].at[:, pl.ds(0, HF)],
                ea_hbm.at[pl.ds(ebt, CH), pl.ds(cid * HF, HF)],
                sem_o[bb]).wait()
            pltpu.make_async_copy(out_v[bb], acc_sh.at[dst_v[bb]],
                                  sem_a[bb]).wait()
            pltpu.make_async_copy(out_v[bb], gacc_sh.at[gdst_v[bb]],
                                  sem_ga[bb]).wait()
        return c

    lax.fori_loop(0, NMB, _meta, 0)

    plsc.subcore_barrier()
    pltpu.sync_copy(acc_sh.at[pl.ds(sid * RPT, RPT)],
                    acc_hbm.at[cid, pl.ds(sid * RPT, RPT)])
    pltpu.sync_copy(gacc_sh.at[pl.ds(sid * G, G)],
                    gacc_hbm.at[cid, pl.ds(sid * G, G)])


def _build_sc_edge():
    mesh = plsc.VectorSubcoreMesh(core_axis_name="c", subcore_axis_name="s")
    return pl.kernel(
        _sc_edge_body,
        out_type=(
            jax.ShapeDtypeStruct((E, OE), jnp.float32),
            jax.ShapeDtypeStruct((NC, AR, ACW), jnp.float32),
            jax.ShapeDtypeStruct((NC, NS * G, ACW), jnp.float32),
        ),
        mesh=mesh,
        compiler_params=pltpu.CompilerParams(use_tc_tiling_on_sc=False,
                                             needs_layout_passes=False),
        scratch_types=(
            [
                pltpu.VMEM((N,), jnp.int32),
                pltpu.VMEM((MB,), jnp.int32),
                pltpu.VMEM((MB,), jnp.int32),
            ]
            + [pltpu.VMEM((CH,), jnp.int32)] * 6
            + [pltpu.VMEM((CH, HF), jnp.float32)] * 4
            + [pltpu.VMEM((CH, ACW), jnp.float32)] * 2
            + [
                pltpu.VMEM((ZR, ACW), jnp.float32),
                pltpu.VMEM_SHARED((AR, ACW), jnp.float32),
                pltpu.VMEM_SHARED((NS * G, ACW), jnp.float32),
            ]
            + [pltpu.SemaphoreType.DMA] * 10
        ),
    )


def _final_body(x_ref, b2_ref, u_ref, acc0_ref, acc1_ref,
                gacc0_ref, gacc1_ref,
                wn_ref, bn_ref, wg_ref, bg_ref, xn_ref, un_ref):
    acc0 = acc0_ref[...]
    acc1 = acc1_ref[...]
    node_cnt = jnp.maximum(acc0[:N, HF:HF + 1], 1.0)
    msg0 = acc0[:N, :HF] / node_cnt
    msg1 = acc1[:N, :HF] / node_cnt

    wn = wn_ref[...]
    u_full = u_ref[...]
    uwn = jnp.dot(u_full, wn[DN + OE:], preferred_element_type=jnp.float32)
    oh = (b2_ref[...] == lax.broadcasted_iota(jnp.int32, (1, G), 1)
          ).astype(jnp.float32)
    xn = (jnp.dot(x_ref[...], wn[:DN], preferred_element_type=jnp.float32)
          + jnp.dot(msg0, wn[DN:DN + HF], preferred_element_type=jnp.float32)
          + jnp.dot(msg1, wn[DN + HF:DN + OE],
                    preferred_element_type=jnp.float32)
          + jnp.dot(oh, uwn, preferred_element_type=jnp.float32)
          + bn_ref[...])
    xn = jnp.maximum(xn, 0.0)
    xn_ref[...] = xn

    gsum = lax.dot_general(oh, xn, (((0,), (0,)), ((), ())),
                           preferred_element_type=jnp.float32)
    gcnt = lax.dot_general(oh, jnp.ones((N, 1), jnp.float32),
                           (((0,), (0,)), ((), ())),
                           preferred_element_type=jnp.float32)
    x_global = gsum / jnp.maximum(gcnt, 1.0)

    ga0 = gacc0_ref[...]
    ga1 = gacc1_ref[...]
    gs0 = ga0[0:G]
    gs1 = ga1[0:G]
    for s in range(1, NS):
        gs0 = gs0 + ga0[s * G:(s + 1) * G]
        gs1 = gs1 + ga1[s * G:(s + 1) * G]
    ecnt = jnp.maximum(gs0[:, HF:HF + 1], 1.0)
    eg0 = gs0[:, :HF] / ecnt
    eg1 = gs1[:, :HF] / ecnt

    wg = wg_ref[...]
    un = (jnp.dot(eg0, wg[:HF], preferred_element_type=jnp.float32)
          + jnp.dot(eg1, wg[HF:OE], preferred_element_type=jnp.float32)
          + jnp.dot(x_global, wg[OE:OE + ON], preferred_element_type=jnp.float32)
          + jnp.dot(u_full, wg[ON + OE:], preferred_element_type=jnp.float32)
          + bg_ref[...])
    un_ref[...] = jnp.maximum(un, 0.0)


@jax.jit
def kernel(x, edge_attr, edge_index, u, batch, W_e, b_e, W_n, b_n, W_g, b_g):
    batch2d = batch.reshape(N, 1)
    src = edge_index[0]
    dst = edge_index[1]

    node_pre = pl.pallas_call(
        _node_pre_body,
        out_shape=jax.ShapeDtypeStruct((NC, N, HF), jnp.float32),
    )(x, batch2d, u, W_e, b_e)

    BE = 8000
    eat = pl.pallas_call(
        _eat_body,
        grid=(E // BE,),
        in_specs=[
            pl.BlockSpec((BE, DE), lambda i: (i, 0)),
            pl.BlockSpec((DN + DE + DU, OE), lambda i: (0, 0)),
        ],
        out_specs=pl.BlockSpec((BE, OE), lambda i: (i, 0)),
        out_shape=jax.ShapeDtypeStruct((E, OE), jnp.float32),
    )(edge_attr, W_e)

    ea, acc, gacc = _build_sc_edge()(node_pre, eat, src, dst, batch)

    xn, un = pl.pallas_call(
        _final_body,
        out_shape=(
            jax.ShapeDtypeStruct((N, ON), jnp.float32),
            jax.ShapeDtypeStruct((G, OU), jnp.float32),
        ),
    )(x, batch2d, u, acc[0], acc[1], gacc[0], gacc[1], W_n, b_n, W_g, b_g)

    return (xn, ea, edge_index, un, batch)
